# baseline (speedup 1.0000x reference)
import functools

import jax
import jax.numpy as jnp
from jax import lax
from jax.experimental import pallas as pl
from jax.experimental.pallas import tpu as pltpu
from jax.experimental.pallas import tpu_sc as plsc

_NC = 2
_NS = 16
_NW = _NC * _NS

_NE = 1000000
_B = 4096 * 50
_D = 64
_BPW = _B // _NW
_CHUNK = 800
_NCHUNK = _BPW // _CHUNK

_HALF = 524288
_TBLK = 1024


def _tc_transpose():
    def body(a_ref, b_ref, out_ref):
        out_ref[:, 0:_D] = jnp.transpose(a_ref[...])
        out_ref[:, _D:128] = jnp.transpose(b_ref[...])

    nblk = _HALF // _TBLK
    last = (_NE - 1) // _TBLK
    return pl.pallas_call(
        body,
        grid=(nblk,),
        in_specs=[
            pl.BlockSpec((_D, _TBLK), lambda j: (0, j)),
            pl.BlockSpec(
                (_D, _TBLK),
                lambda j, n=nblk, m=last: (0, jnp.minimum(j + n, m))),
        ],
        out_specs=pl.BlockSpec((_TBLK, 128), lambda j: (j, 0)),
        out_shape=jax.ShapeDtypeStruct((_HALF, 128), jnp.float32),
    )


_transpose = _tc_transpose()


def _make_gather():
    mesh = plsc.VectorSubcoreMesh(
        core_axis_name="c",
        subcore_axis_name="s",
        num_cores=_NC,
        num_subcores=_NS,
    )

    @functools.partial(
        pl.kernel,
        out_type=jax.ShapeDtypeStruct((_B, _D), jnp.float32),
        mesh=mesh,
        scratch_types=[
            pltpu.VMEM((_CHUNK,), jnp.int32),
            pltpu.VMEM((_CHUNK,), jnp.int32),
            pltpu.VMEM((_CHUNK, _D), jnp.float32),
            pltpu.VMEM((_CHUNK, _D), jnp.float32),
            pltpu.SemaphoreType.DMA,
            pltpu.SemaphoreType.DMA,
            pltpu.SemaphoreType.DMA,
            pltpu.SemaphoreType.DMA,
        ],
        compiler_params=pltpu.CompilerParams(use_tc_tiling_on_sc=False),
    )
    def gather(idx_hbm, table_hbm, out_hbm,
               idx0, idx1, rows0, rows1, gsem0, gsem1, osem0, osem1):
        wid = lax.axis_index("s") * _NC + lax.axis_index("c")
        base = wid * _BPW
        idx = (idx0, idx1)
        rows = (rows0, rows1)
        gsem = (gsem0, gsem1)
        osem = (osem0, osem1)

        gat = [None, None]
        outcp = [None, None]
        pltpu.sync_copy(idx_hbm.at[wid, 0], idx0)
        gat[0] = pltpu.async_copy(table_hbm.at[idx0], rows0, gsem0)
        for g in range(_NCHUNK):
            b = g & 1
            nb = 1 - b
            if g + 1 < _NCHUNK:
                pltpu.sync_copy(idx_hbm.at[wid, g + 1], idx[nb])
                if outcp[nb] is not None:
                    outcp[nb].wait()
                gat[nb] = pltpu.async_copy(
                    table_hbm.at[idx[nb]], rows[nb], gsem[nb])
            gat[b].wait()
            outcp[b] = pltpu.async_copy(
                rows[b], out_hbm.at[pl.ds(base + g * _CHUNK, _CHUNK)],
                osem[b])
        outcp[0].wait()
        outcp[1].wait()

    return gather


_gather = _make_gather()


def kernel(x, weight):
    wt = weight.T
    table = _transpose(wt, wt).reshape(2 * _HALF, _D)
    xi = jnp.where(x < _HALF, x * 2, (x - _HALF) * 2 + 1)
    idx = xi.reshape(_NW, _NCHUNK, _CHUNK)
    out = _gather(idx, table)
    return out.reshape(x.shape[0], x.shape[1], _D)


# speedup vs baseline: 1.4390x; 1.4390x over previous
import functools

import jax
import jax.numpy as jnp
from jax import lax
from jax.experimental import pallas as pl
from jax.experimental.pallas import tpu as pltpu
from jax.experimental.pallas import tpu_sc as plsc

_NC = 2
_NS = 16
_NW = _NC * _NS

_NE = 1000000
_B = 4096 * 50
_D = 64
_BPW = _B // _NW
_CHUNK = 800
_NCHUNK = _BPW // _CHUNK

_HALF = 524288
_TBLK = 4096


def _tc_transpose():
    def body(a_ref, b_ref, out_ref):
        out_ref[:, 0:_D] = jnp.transpose(a_ref[...])
        out_ref[:, _D:128] = jnp.transpose(b_ref[...])

    nblk = _HALF // _TBLK
    last = (_NE - 1) // _TBLK
    return pl.pallas_call(
        body,
        grid=(nblk,),
        in_specs=[
            pl.BlockSpec((_D, _TBLK), lambda j: (0, j)),
            pl.BlockSpec(
                (_D, _TBLK),
                lambda j, n=nblk, m=last: (0, jnp.minimum(j + n, m))),
        ],
        out_specs=pl.BlockSpec((_TBLK, 128), lambda j: (j, 0)),
        out_shape=jax.ShapeDtypeStruct((_HALF, 128), jnp.float32),
    )


_transpose = _tc_transpose()


def _make_gather():
    mesh = plsc.VectorSubcoreMesh(
        core_axis_name="c",
        subcore_axis_name="s",
        num_cores=_NC,
        num_subcores=_NS,
    )

    @functools.partial(
        pl.kernel,
        out_type=jax.ShapeDtypeStruct((_B, _D), jnp.float32),
        mesh=mesh,
        scratch_types=[
            pltpu.VMEM((_CHUNK,), jnp.int32),
            pltpu.VMEM((_CHUNK,), jnp.int32),
            pltpu.VMEM((_CHUNK, _D), jnp.float32),
            pltpu.VMEM((_CHUNK, _D), jnp.float32),
            pltpu.SemaphoreType.DMA,
            pltpu.SemaphoreType.DMA,
            pltpu.SemaphoreType.DMA,
            pltpu.SemaphoreType.DMA,
        ],
        compiler_params=pltpu.CompilerParams(use_tc_tiling_on_sc=False),
    )
    def gather(idx_hbm, table_hbm, out_hbm,
               idx0, idx1, rows0, rows1, gsem0, gsem1, osem0, osem1):
        wid = lax.axis_index("s") * _NC + lax.axis_index("c")
        base = wid * _BPW
        idx = (idx0, idx1)
        rows = (rows0, rows1)
        gsem = (gsem0, gsem1)
        osem = (osem0, osem1)

        gat = [None, None]
        outcp = [None, None]
        pltpu.sync_copy(idx_hbm.at[wid, 0], idx0)
        gat[0] = pltpu.async_copy(table_hbm.at[idx0], rows0, gsem0)
        for g in range(_NCHUNK):
            b = g & 1
            nb = 1 - b
            if g + 1 < _NCHUNK:
                pltpu.sync_copy(idx_hbm.at[wid, g + 1], idx[nb])
                if outcp[nb] is not None:
                    outcp[nb].wait()
                gat[nb] = pltpu.async_copy(
                    table_hbm.at[idx[nb]], rows[nb], gsem[nb])
            gat[b].wait()
            outcp[b] = pltpu.async_copy(
                rows[b], out_hbm.at[pl.ds(base + g * _CHUNK, _CHUNK)],
                osem[b])
        outcp[0].wait()
        outcp[1].wait()

    return gather


_gather = _make_gather()


def kernel(x, weight):
    wt = weight.T
    table = _transpose(wt, wt).reshape(2 * _HALF, _D)
    xi = jnp.where(x < _HALF, x * 2, (x - _HALF) * 2 + 1)
    idx = xi.reshape(_NW, _NCHUNK, _CHUNK)
    out = _gather(idx, table)
    return out.reshape(x.shape[0], x.shape[1], _D)


# speedup vs baseline: 1.5562x; 1.0814x over previous
import functools

import jax
import jax.numpy as jnp
from jax import lax
from jax.experimental import pallas as pl
from jax.experimental.pallas import tpu as pltpu
from jax.experimental.pallas import tpu_sc as plsc

_NC = 2
_NS = 16
_NW = _NC * _NS

_NE = 1000000
_B = 4096 * 50
_D = 64
_BPW = _B // _NW
_CHUNK = 800
_NCHUNK = _BPW // _CHUNK

_HALF = 524288
_TBLK = 8192


def _tc_transpose():
    def body(a_ref, b_ref, out_ref):
        out_ref[:, 0:_D] = jnp.transpose(a_ref[...])
        out_ref[:, _D:128] = jnp.transpose(b_ref[...])

    nblk = _HALF // _TBLK
    last = (_NE - 1) // _TBLK
    return pl.pallas_call(
        body,
        grid=(nblk,),
        in_specs=[
            pl.BlockSpec((_D, _TBLK), lambda j: (0, j)),
            pl.BlockSpec(
                (_D, _TBLK),
                lambda j, n=nblk, m=last: (0, jnp.minimum(j + n, m))),
        ],
        out_specs=pl.BlockSpec((_TBLK, 128), lambda j: (j, 0)),
        out_shape=jax.ShapeDtypeStruct((_HALF, 128), jnp.float32),
    )


_transpose = _tc_transpose()


def _make_gather():
    mesh = plsc.VectorSubcoreMesh(
        core_axis_name="c",
        subcore_axis_name="s",
        num_cores=_NC,
        num_subcores=_NS,
    )

    @functools.partial(
        pl.kernel,
        out_type=jax.ShapeDtypeStruct((_B, _D), jnp.float32),
        mesh=mesh,
        scratch_types=[
            pltpu.VMEM((_CHUNK,), jnp.int32),
            pltpu.VMEM((_CHUNK,), jnp.int32),
            pltpu.VMEM((_CHUNK, _D), jnp.float32),
            pltpu.VMEM((_CHUNK, _D), jnp.float32),
            pltpu.SemaphoreType.DMA,
            pltpu.SemaphoreType.DMA,
            pltpu.SemaphoreType.DMA,
            pltpu.SemaphoreType.DMA,
        ],
        compiler_params=pltpu.CompilerParams(use_tc_tiling_on_sc=False),
    )
    def gather(idx_hbm, table_hbm, out_hbm,
               idx0, idx1, rows0, rows1, gsem0, gsem1, osem0, osem1):
        wid = lax.axis_index("s") * _NC + lax.axis_index("c")
        base = wid * _BPW
        idx = (idx0, idx1)
        rows = (rows0, rows1)
        gsem = (gsem0, gsem1)
        osem = (osem0, osem1)

        gat = [None, None]
        outcp = [None, None]
        pltpu.sync_copy(idx_hbm.at[wid, 0], idx0)
        gat[0] = pltpu.async_copy(table_hbm.at[idx0], rows0, gsem0)
        for g in range(_NCHUNK):
            b = g & 1
            nb = 1 - b
            if g + 1 < _NCHUNK:
                pltpu.sync_copy(idx_hbm.at[wid, g + 1], idx[nb])
                if outcp[nb] is not None:
                    outcp[nb].wait()
                gat[nb] = pltpu.async_copy(
                    table_hbm.at[idx[nb]], rows[nb], gsem[nb])
            gat[b].wait()
            outcp[b] = pltpu.async_copy(
                rows[b], out_hbm.at[pl.ds(base + g * _CHUNK, _CHUNK)],
                osem[b])
        outcp[0].wait()
        outcp[1].wait()

    return gather


_gather = _make_gather()


def kernel(x, weight):
    wt = weight.T
    table = _transpose(wt, wt).reshape(2 * _HALF, _D)
    xi = jnp.where(x < _HALF, x * 2, (x - _HALF) * 2 + 1)
    idx = xi.reshape(_NW, _NCHUNK, _CHUNK)
    out = _gather(idx, table)
    return out.reshape(x.shape[0], x.shape[1], _D)


# speedup vs baseline: 1.6125x; 1.0362x over previous
import functools

import jax
import jax.numpy as jnp
from jax import lax
from jax.experimental import pallas as pl
from jax.experimental.pallas import tpu as pltpu
from jax.experimental.pallas import tpu_sc as plsc

_NC = 2
_NS = 16
_NW = _NC * _NS

_NE = 1000000
_B = 4096 * 50
_D = 64
_BPW = _B // _NW
_CHUNK = 800
_NCHUNK = _BPW // _CHUNK

_HALF = 524288
_TBLK = 16384


def _tc_transpose():
    def body(a_ref, b_ref, out_ref):
        out_ref[:, 0:_D] = jnp.transpose(a_ref[...])
        out_ref[:, _D:128] = jnp.transpose(b_ref[...])

    nblk = _HALF // _TBLK
    last = (_NE - 1) // _TBLK
    return pl.pallas_call(
        body,
        grid=(nblk,),
        in_specs=[
            pl.BlockSpec((_D, _TBLK), lambda j: (0, j)),
            pl.BlockSpec(
                (_D, _TBLK),
                lambda j, n=nblk, m=last: (0, jnp.minimum(j + n, m))),
        ],
        out_specs=pl.BlockSpec((_TBLK, 128), lambda j: (j, 0)),
        out_shape=jax.ShapeDtypeStruct((_HALF, 128), jnp.float32),
    )


_transpose = _tc_transpose()


def _make_gather():
    mesh = plsc.VectorSubcoreMesh(
        core_axis_name="c",
        subcore_axis_name="s",
        num_cores=_NC,
        num_subcores=_NS,
    )

    @functools.partial(
        pl.kernel,
        out_type=jax.ShapeDtypeStruct((_B, _D), jnp.float32),
        mesh=mesh,
        scratch_types=[
            pltpu.VMEM((_CHUNK,), jnp.int32),
            pltpu.VMEM((_CHUNK,), jnp.int32),
            pltpu.VMEM((_CHUNK, _D), jnp.float32),
            pltpu.VMEM((_CHUNK, _D), jnp.float32),
            pltpu.SemaphoreType.DMA,
            pltpu.SemaphoreType.DMA,
            pltpu.SemaphoreType.DMA,
            pltpu.SemaphoreType.DMA,
        ],
        compiler_params=pltpu.CompilerParams(use_tc_tiling_on_sc=False),
    )
    def gather(idx_hbm, table_hbm, out_hbm,
               idx0, idx1, rows0, rows1, gsem0, gsem1, osem0, osem1):
        wid = lax.axis_index("s") * _NC + lax.axis_index("c")
        base = wid * _BPW
        idx = (idx0, idx1)
        rows = (rows0, rows1)
        gsem = (gsem0, gsem1)
        osem = (osem0, osem1)

        gat = [None, None]
        outcp = [None, None]
        pltpu.sync_copy(idx_hbm.at[wid, 0], idx0)
        gat[0] = pltpu.async_copy(table_hbm.at[idx0], rows0, gsem0)
        for g in range(_NCHUNK):
            b = g & 1
            nb = 1 - b
            if g + 1 < _NCHUNK:
                pltpu.sync_copy(idx_hbm.at[wid, g + 1], idx[nb])
                if outcp[nb] is not None:
                    outcp[nb].wait()
                gat[nb] = pltpu.async_copy(
                    table_hbm.at[idx[nb]], rows[nb], gsem[nb])
            gat[b].wait()
            outcp[b] = pltpu.async_copy(
                rows[b], out_hbm.at[pl.ds(base + g * _CHUNK, _CHUNK)],
                osem[b])
        outcp[0].wait()
        outcp[1].wait()

    return gather


_gather = _make_gather()


def kernel(x, weight):
    wt = weight.T
    table = _transpose(wt, wt).reshape(2 * _HALF, _D)
    xi = jnp.where(x < _HALF, x * 2, (x - _HALF) * 2 + 1)
    idx = xi.reshape(_NW, _NCHUNK, _CHUNK)
    out = _gather(idx, table)
    return out.reshape(x.shape[0], x.shape[1], _D)
